# baseline (speedup 1.0000x reference)
import functools

import jax
import jax.numpy as jnp
from jax import lax
from jax.experimental import pallas as pl
from jax.experimental.pallas import tpu as pltpu
from jax.experimental.pallas import tpu_sc as plsc

_NUM_CORES = 2
_NUM_SUBCORES = 16
_NUM_WORKERS = _NUM_CORES * _NUM_SUBCORES
_L = 16


@functools.lru_cache(maxsize=None)
def _make_sc(V, D, B):
    assert B % (_NUM_WORKERS * _L) == 0
    b_per_w = B // _NUM_WORKERS
    mesh = plsc.VectorSubcoreMesh(core_axis_name="c", subcore_axis_name="s")

    @functools.partial(
        pl.kernel,
        mesh=mesh,
        out_type=jax.ShapeDtypeStruct((B, D), jnp.float32),
        scratch_types=[
            pltpu.VMEM_SHARED((8 * V * V, D), jnp.float32),
            pltpu.VMEM((b_per_w,), jnp.int32),
            pltpu.SemaphoreType.DMA,
        ],
    )
    def k(pairs_hbm, idx_hbm, out_hbm, pairs_sh, idx_v, sem):
        sid = lax.axis_index("s")
        wid = sid * _NUM_CORES + lax.axis_index("c")
        base = wid * b_per_w

        nblk = V * V
        sl0 = pl.ds(sid * 8, 8)
        pltpu.sync_copy(pairs_hbm.at[sl0], pairs_sh.at[sl0])

        @pl.when(sid < nblk - _NUM_SUBCORES)
        def _stage_rest():
            sl1 = pl.ds((_NUM_SUBCORES + sid) * 8, 8)
            pltpu.sync_copy(pairs_hbm.at[sl1], pairs_sh.at[sl1])

        pltpu.sync_copy(idx_hbm.at[pl.ds(base, b_per_w)], idx_v)
        plsc.subcore_barrier()
        handles = []
        for g in range(b_per_w // _L):
            vec = idx_v[pl.ds(g * _L, _L)]
            for j in range(0, _L, 2):
                s = (vec[j] * V + vec[j + 1]) * 8
                handles.append(pltpu.async_copy(
                    pairs_sh.at[pl.ds(s, 2)],
                    out_hbm.at[pl.ds(base + g * _L + j, 2)],
                    sem))
        for h in handles:
            h.wait()

    return k


def kernel(disease_embeds, disease_indices):
    V, D = disease_embeds.shape
    (B,) = disease_indices.shape
    idx = disease_indices.astype(jnp.int32)
    pairs = jnp.stack(
        [jnp.repeat(disease_embeds, V, axis=0),
         jnp.tile(disease_embeds, (V, 1))], axis=1)
    pairs_pad = jnp.pad(pairs, ((0, 0), (0, 6), (0, 0))).reshape(
        8 * V * V, D)
    return _make_sc(V, D, B)(pairs_pad, idx)


# speedup vs baseline: 1.0241x; 1.0241x over previous
import functools

import jax
import jax.numpy as jnp
from jax import lax
from jax.experimental import pallas as pl
from jax.experimental.pallas import tpu as pltpu
from jax.experimental.pallas import tpu_sc as plsc

_NUM_CORES = 2
_NUM_SUBCORES = 16
_NUM_WORKERS = _NUM_CORES * _NUM_SUBCORES
_L = 16


@functools.lru_cache(maxsize=None)
def _make_sc(V, D, B):
    assert B % (_NUM_WORKERS * _L) == 0
    b_per_w = B // _NUM_WORKERS
    mesh = plsc.VectorSubcoreMesh(core_axis_name="c", subcore_axis_name="s")

    @functools.partial(
        pl.kernel,
        mesh=mesh,
        out_type=jax.ShapeDtypeStruct((B, D), jnp.float32),
        scratch_types=[
            pltpu.VMEM_SHARED((8 * V * V, D), jnp.float32),
            pltpu.VMEM((b_per_w,), jnp.int32),
            pltpu.SemaphoreType.DMA,
        ],
    )
    def k(pairs_hbm, idx_hbm, out_hbm, pairs_sh, idx_v, sem):
        sid = lax.axis_index("s")
        wid = sid * _NUM_CORES + lax.axis_index("c")
        base = wid * b_per_w

        @pl.when(sid == 0)
        def _stage():
            pltpu.sync_copy(pairs_hbm, pairs_sh)

        pltpu.sync_copy(idx_hbm.at[pl.ds(base, b_per_w)], idx_v)
        plsc.subcore_barrier()
        handles = []
        for g in range(b_per_w // _L):
            vec = idx_v[pl.ds(g * _L, _L)]
            for j in range(0, _L, 2):
                s = (vec[j] * V + vec[j + 1]) * 8
                handles.append(pltpu.async_copy(
                    pairs_sh.at[pl.ds(s, 2)],
                    out_hbm.at[pl.ds(base + g * _L + j, 2)],
                    sem))
        for h in handles:
            h.wait()

    return k


def kernel(disease_embeds, disease_indices):
    V, D = disease_embeds.shape
    (B,) = disease_indices.shape
    idx = disease_indices.astype(jnp.int32)
    pairs = jnp.stack(
        [jnp.repeat(disease_embeds, V, axis=0),
         jnp.tile(disease_embeds, (V, 1))], axis=1)
    pairs_pad = jnp.pad(pairs, ((0, 0), (0, 6), (0, 0))).reshape(
        8 * V * V, D)
    return _make_sc(V, D, B)(pairs_pad, idx)
